# baseline (speedup 1.0000x reference)
import functools

import jax
import jax.numpy as jnp
from jax.experimental import pallas as pl
from jax.experimental import pallas as _pl
from jax.experimental.pallas import tpu as pltpu


def _fused_body(idx_ref, x_ref, W_ref, b_ref, Wd_ref, bd_ref, z_ref, y_ref,
                W_sc, *, E, L):
    i = pl.program_id(0)

    @pl.when(i == 0)
    def _():
        for e in range(E):
            W_sc[:, e * L:(e + 1) * L] = W_ref[e].astype(jnp.bfloat16)

    x_t = x_ref[...].astype(jnp.bfloat16)
    ids = idx_ref[...][:, :1]
    big = jnp.dot(x_t, W_sc[...], preferred_element_type=jnp.float32)
    big = big + b_ref[...]
    acc = jnp.zeros(z_ref.shape, dtype=jnp.float32)
    for e in range(E):
        acc = jnp.where(ids == e, big[:, e * L:(e + 1) * L], acc)
    z_ref[...] = acc
    y_ref[...] = jnp.dot(acc, Wd_ref[...], preferred_element_type=jnp.float32) + bd_ref[0]


def kernel(x, individual_idx, W_enc, b_enc, W_dec, b_dec):
    N, D = x.shape
    E, _, L = W_enc.shape
    TN = 1024
    nb = N // TN
    idx8 = jnp.broadcast_to(individual_idx.astype(jnp.int32)[:, None], (N, 8))
    b_all = b_enc.reshape(1, E * L)

    z, y = pl.pallas_call(
        functools.partial(_fused_body, E=E, L=L),
        grid=(nb,),
        in_specs=[
            pl.BlockSpec((TN, 8), lambda i: (i, 0)),
            pl.BlockSpec((TN, D), lambda i: (i, 0)),
            pl.BlockSpec((E, D, L), lambda i: (0, 0, 0)),
            pl.BlockSpec((1, E * L), lambda i: (0, 0)),
            pl.BlockSpec((L, 1), lambda i: (0, 0)),
            pl.BlockSpec((1,), lambda i: (0,)),
        ],
        out_specs=[
            pl.BlockSpec((TN, L), lambda i: (i, 0)),
            pl.BlockSpec((TN, 1), lambda i: (i, 0)),
        ],
        out_shape=[
            jax.ShapeDtypeStruct((N, L), jnp.float32),
            jax.ShapeDtypeStruct((N, 1), jnp.float32),
        ],
        scratch_shapes=[pltpu.VMEM((D, E * L), jnp.bfloat16)],
    )(idx8, x, W_enc, b_all, W_dec, b_dec)
    return (y, z)


# speedup vs baseline: 1.1293x; 1.1293x over previous
import functools

import jax
import jax.numpy as jnp
from jax.experimental import pallas as pl
from jax.experimental import pallas as _pl
from jax.experimental.pallas import tpu as pltpu


def _fused_body(idx_ref, x_ref, W_ref, b_ref, Wd_ref, bd_ref, z_ref, y_ref,
                W_sc, *, E, L):
    i = pl.program_id(0)

    @pl.when(i == 0)
    def _():
        for e in range(E):
            W_sc[:, e * L:(e + 1) * L] = W_ref[e].astype(jnp.bfloat16)

    x_t = x_ref[...].astype(jnp.bfloat16)
    ids = idx_ref[...]
    big = jnp.dot(x_t, W_sc[...], preferred_element_type=jnp.float32)
    big = big + b_ref[...]
    acc = jnp.zeros(z_ref.shape, dtype=jnp.float32)
    for e in range(E):
        acc = jnp.where(ids == e, big[:, e * L:(e + 1) * L], acc)
    z_ref[...] = acc
    yv = jnp.dot(acc, Wd_ref[...], preferred_element_type=jnp.float32) + bd_ref[0]
    y_ref[...] = yv.reshape(yv.shape[0])


def kernel(x, individual_idx, W_enc, b_enc, W_dec, b_dec):
    N, D = x.shape
    E, _, L = W_enc.shape
    TN = 1024
    nb = N // TN
    idx2 = individual_idx.astype(jnp.int32).reshape(N, 1)
    b_all = b_enc.reshape(1, E * L)

    z, y = pl.pallas_call(
        functools.partial(_fused_body, E=E, L=L),
        grid=(nb,),
        in_specs=[
            pl.BlockSpec((TN, 1), lambda i: (i, 0)),
            pl.BlockSpec((TN, D), lambda i: (i, 0)),
            pl.BlockSpec((E, D, L), lambda i: (0, 0, 0)),
            pl.BlockSpec((1, E * L), lambda i: (0, 0)),
            pl.BlockSpec((L, 1), lambda i: (0, 0)),
            pl.BlockSpec((1,), lambda i: (0,)),
        ],
        out_specs=[
            pl.BlockSpec((TN, L), lambda i: (i, 0)),
            pl.BlockSpec((TN,), lambda i: (i,)),
        ],
        out_shape=[
            jax.ShapeDtypeStruct((N, L), jnp.float32),
            jax.ShapeDtypeStruct((N,), jnp.float32),
        ],
        scratch_shapes=[pltpu.VMEM((D, E * L), jnp.bfloat16)],
    )(idx2, x, W_enc, b_all, W_dec, b_dec)
    return (y.reshape(N, 1), z)


# speedup vs baseline: 1.2366x; 1.0950x over previous
import functools

import jax
import jax.numpy as jnp
from jax import lax
from jax.experimental import pallas as pl
from jax.experimental.pallas import tpu as pltpu


def _fused_body(idx_ref, x_ref, W_ref, b_ref, Wd_ref, bd_ref, z_ref, y_ref,
                W_sc, *, E, L):
    i = pl.program_id(0)

    @pl.when(i == 0)
    def _():
        for e in range(E):
            W_sc[:, e * L:(e + 1) * L] = W_ref[e].astype(jnp.bfloat16)

    x_t = x_ref[...].astype(jnp.bfloat16)
    TN = x_t.shape[0]
    ids_row = idx_ref[0]
    oh = (jnp.broadcast_to(ids_row, (E, TN))
          == lax.broadcasted_iota(jnp.int32, (E, TN), 0)).astype(jnp.float32)
    oht = oh.T
    big = jnp.dot(x_t, W_sc[...], preferred_element_type=jnp.float32)
    big = big + b_ref[...]
    acc = jnp.zeros(z_ref.shape, dtype=jnp.float32)
    for e in range(E):
        acc = acc + oht[:, e:e + 1] * big[:, e * L:(e + 1) * L]
    z_ref[...] = acc
    yv = jnp.dot(acc, Wd_ref[...], preferred_element_type=jnp.float32) + bd_ref[0]
    y_ref[...] = yv.reshape(yv.shape[0])


def kernel(x, individual_idx, W_enc, b_enc, W_dec, b_dec):
    N, D = x.shape
    E, _, L = W_enc.shape
    TN = 1024
    nb = N // TN
    idx3 = individual_idx.astype(jnp.int32).reshape(nb, 1, TN)
    b_all = b_enc.reshape(1, E * L)

    z, y = pl.pallas_call(
        functools.partial(_fused_body, E=E, L=L),
        grid=(nb,),
        in_specs=[
            pl.BlockSpec((1, 1, TN), lambda i: (i, 0, 0)),
            pl.BlockSpec((TN, D), lambda i: (i, 0)),
            pl.BlockSpec((E, D, L), lambda i: (0, 0, 0)),
            pl.BlockSpec((1, E * L), lambda i: (0, 0)),
            pl.BlockSpec((L, 1), lambda i: (0, 0)),
            pl.BlockSpec((1,), lambda i: (0,)),
        ],
        out_specs=[
            pl.BlockSpec((TN, L), lambda i: (i, 0)),
            pl.BlockSpec((TN,), lambda i: (i,)),
        ],
        out_shape=[
            jax.ShapeDtypeStruct((N, L), jnp.float32),
            jax.ShapeDtypeStruct((N,), jnp.float32),
        ],
        scratch_shapes=[pltpu.VMEM((D, E * L), jnp.bfloat16)],
    )(idx3, x, W_enc, b_all, W_dec, b_dec)
    return (y.reshape(N, 1), z)
